# baseline (speedup 1.0000x reference)
import functools

import jax
import jax.numpy as jnp
from jax import lax
from jax.experimental import pallas as pl
from jax.experimental.pallas import tpu as pltpu
from jax.experimental.pallas import tpu_sc as plsc

_N = 10000
_E = 320000
_NP = 10240
_NC = 2
_NS = 16
_NW = _NC * _NS
_CHUNK = 80
_NCHUNK = 128
_EW = _NCHUNK * _CHUNK
_EPAD = _NW * _EW
_RSUB = _NP // _NS
_WROWS = _CHUNK
_WB = _RSUB // _WROWS


def _sc_mesh():
    return plsc.VectorSubcoreMesh(core_axis_name="c", subcore_axis_name="s")


def _make_agg(D):
    @functools.partial(
        pl.kernel,
        out_type=jax.ShapeDtypeStruct((_NC, _NP, D), jnp.float32),
        mesh=_sc_mesh(),
        scratch_types=[
            pltpu.VMEM((_CHUNK,), jnp.int32),
            pltpu.VMEM((_CHUNK,), jnp.int32),
            pltpu.VMEM((_CHUNK,), jnp.int32),
            pltpu.VMEM((_CHUNK,), jnp.int32),
            pltpu.VMEM((_CHUNK,), jnp.int32),
            pltpu.VMEM((_CHUNK,), jnp.int32),
            pltpu.VMEM((_CHUNK,), jnp.int32),
            pltpu.VMEM((_CHUNK,), jnp.int32),
            pltpu.VMEM((_CHUNK, D), jnp.float32),
            pltpu.VMEM((_CHUNK, D), jnp.float32),
            pltpu.VMEM_SHARED((_NP, D), jnp.float32),
            pltpu.SemaphoreType.DMA,
            pltpu.SemaphoreType.DMA,
            pltpu.SemaphoreType.DMA,
            pltpu.SemaphoreType.DMA,
            pltpu.SemaphoreType.DMA,
            pltpu.SemaphoreType.DMA,
        ],
    )
    def agg(hp_hbm, src_hbm, dst_hbm, out_hbm, s0, s1, s2, s3, d0, d1, d2,
            d3, ma, mb, acc, semi0, semi1, semi2, semi3, sema, semb):
        c = lax.axis_index("c")
        s = lax.axis_index("s")
        base = (c * _NS + s) * _EW
        sv = [s0, s1, s2, s3]
        dv = [d0, d1, d2, d3]
        semi = [semi0, semi1, semi2, semi3]

        def ldx(i, j):
            off = base + i * _CHUNK
            ca = pltpu.make_async_copy(src_hbm.at[pl.ds(off, _CHUNK)], sv[j],
                                       semi[j])
            cb = pltpu.make_async_copy(dst_hbm.at[pl.ds(off, _CHUNK)], dv[j],
                                       semi[j])
            return ca, cb

        def ldx_start(i, j):
            ca, cb = ldx(i, j)
            ca.start()
            cb.start()

        def ldx_wait(i, j):
            ca, cb = ldx(i, j)
            ca.wait()
            cb.wait()

        def gather(j, buf, sem):
            return pltpu.make_async_copy(hp_hbm.at[sv[j]], buf, sem)

        ldx_start(0, 0)
        ldx_start(1, 1)
        ldx_start(2, 2)

        @pl.loop(0, _WROWS)
        def _zero(i):
            for j in range(D // 16):
                ma[i, pl.ds(j * 16, 16)] = jnp.zeros((16,), jnp.float32)

        @pl.loop(0, _WB)
        def _clear(k):
            pltpu.sync_copy(ma, acc.at[pl.ds(s * _RSUB + k * _WROWS, _WROWS)])

        plsc.subcore_barrier()

        ldx_wait(0, 0)
        gather(0, ma, sema).start()

        @pl.loop(0, _NCHUNK // 4 - 1)
        def _edges(t):
            i = t * 4
            ldx_wait(i + 1, 1)
            gather(1, mb, semb).start()
            ldx_start(i + 3, 3)
            gather(0, ma, sema).wait()
            pltpu.sync_copy(ma, acc.at[dv[0]], add=True)
            ldx_wait(i + 2, 2)
            gather(2, ma, sema).start()
            ldx_start(i + 4, 0)
            gather(1, mb, semb).wait()
            pltpu.sync_copy(mb, acc.at[dv[1]], add=True)
            ldx_wait(i + 3, 3)
            gather(3, mb, semb).start()
            ldx_start(i + 5, 1)
            gather(2, ma, sema).wait()
            pltpu.sync_copy(ma, acc.at[dv[2]], add=True)
            ldx_wait(i + 4, 0)
            gather(0, ma, sema).start()
            gather(3, mb, semb).wait()
            pltpu.sync_copy(mb, acc.at[dv[3]], add=True)
            ldx_start(i + 6, 2)

        i0 = _NCHUNK - 4
        ldx_wait(i0 + 1, 1)
        gather(1, mb, semb).start()
        ldx_start(i0 + 3, 3)
        gather(0, ma, sema).wait()
        pltpu.sync_copy(ma, acc.at[dv[0]], add=True)
        ldx_wait(i0 + 2, 2)
        gather(2, ma, sema).start()
        gather(1, mb, semb).wait()
        pltpu.sync_copy(mb, acc.at[dv[1]], add=True)
        ldx_wait(i0 + 3, 3)
        gather(3, mb, semb).start()
        gather(2, ma, sema).wait()
        pltpu.sync_copy(ma, acc.at[dv[2]], add=True)
        gather(3, mb, semb).wait()
        pltpu.sync_copy(mb, acc.at[dv[3]], add=True)

        plsc.subcore_barrier()

        @pl.loop(0, _WB)
        def _writeback(k):
            r0 = s * _RSUB + k * _WROWS
            pltpu.sync_copy(acc.at[pl.ds(r0, _WROWS)],
                            out_hbm.at[c, pl.ds(r0, _WROWS)])

    return agg


def _make_deg():
    D = 16

    @functools.partial(
        pl.kernel,
        out_type=jax.ShapeDtypeStruct((_NC, _NP, D), jnp.float32),
        mesh=_sc_mesh(),
        scratch_types=[
            pltpu.VMEM((_CHUNK,), jnp.int32),
            pltpu.VMEM((_CHUNK,), jnp.int32),
            pltpu.VMEM((_CHUNK,), jnp.int32),
            pltpu.VMEM((_CHUNK,), jnp.int32),
            pltpu.VMEM((_CHUNK, D), jnp.float32),
            pltpu.VMEM((_WROWS, D), jnp.float32),
            pltpu.VMEM_SHARED((_NP, D), jnp.float32),
            pltpu.SemaphoreType.DMA,
            pltpu.SemaphoreType.DMA,
            pltpu.SemaphoreType.DMA,
            pltpu.SemaphoreType.DMA,
            pltpu.SemaphoreType.DMA,
            pltpu.SemaphoreType.DMA,
        ],
    )
    def deg(dst_hbm, out_hbm, d0, d1, d2, d3, onesv, zv, acc, semi0, semi1,
            semi2, semi3, sema, semb):
        c = lax.axis_index("c")
        s = lax.axis_index("s")
        base = (c * _NS + s) * _EW
        dv = [d0, d1, d2, d3]
        semi = [semi0, semi1, semi2, semi3]

        def ldx(i, j):
            off = base + i * _CHUNK
            return pltpu.make_async_copy(dst_hbm.at[pl.ds(off, _CHUNK)],
                                         dv[j], semi[j])

        def scat(j, sem):
            return pltpu.make_async_copy(onesv, acc.at[dv[j]], sem)

        ldx(0, 0).start()
        ldx(1, 1).start()
        ldx(2, 2).start()

        @pl.loop(0, _WROWS)
        def _zero(i):
            zv[i, pl.ds(0, 16)] = jnp.zeros((16,), jnp.float32)

        @pl.loop(0, _CHUNK)
        def _fill(i):
            onesv[i, pl.ds(0, 16)] = jnp.full((16,), 1.0, jnp.float32)

        @pl.loop(0, _WB)
        def _clear(k):
            pltpu.sync_copy(zv, acc.at[pl.ds(s * _RSUB + k * _WROWS, _WROWS)])

        plsc.subcore_barrier()

        ldx(0, 0).wait()
        scat(0, sema).start(add=True)

        @pl.loop(0, _NCHUNK // 4 - 1)
        def _edges(t):
            i = t * 4
            ldx(i + 1, 1).wait()
            scat(1, semb).start(add=True)
            ldx(i + 3, 3).start()
            scat(0, sema).wait()
            ldx(i + 2, 2).wait()
            scat(2, sema).start(add=True)
            ldx(i + 4, 0).start()
            scat(1, semb).wait()
            ldx(i + 3, 3).wait()
            scat(3, semb).start(add=True)
            ldx(i + 5, 1).start()
            scat(2, sema).wait()
            ldx(i + 4, 0).wait()
            scat(0, sema).start(add=True)
            scat(3, semb).wait()
            ldx(i + 6, 2).start()

        i0 = _NCHUNK - 4
        ldx(i0 + 1, 1).wait()
        scat(1, semb).start(add=True)
        ldx(i0 + 3, 3).start()
        scat(0, sema).wait()
        ldx(i0 + 2, 2).wait()
        scat(2, sema).start(add=True)
        scat(1, semb).wait()
        ldx(i0 + 3, 3).wait()
        scat(3, semb).start(add=True)
        scat(2, sema).wait()
        scat(3, semb).wait()

        plsc.subcore_barrier()

        @pl.loop(0, _WB)
        def _writeback(k):
            r0 = s * _RSUB + k * _WROWS
            pltpu.sync_copy(acc.at[pl.ds(r0, _WROWS)],
                            out_hbm.at[c, pl.ds(r0, _WROWS)])

    return deg


_BR = 512
_GRID = _NP // _BR


def _dinv(degp_blk):
    deg = degp_blk[0, :, 0] + degp_blk[1, :, 0] + 1.0
    return lax.rsqrt(deg)


def _mm(a, b):
    return jnp.dot(a, b, preferred_element_type=jnp.float32,
                   precision=lax.Precision.HIGHEST)


def _first_layer_pre(degp_ref, x_ref, w_ref, hp_ref):
    di = _dinv(degp_ref[...])
    hp_ref[...] = _mm(x_ref[...], w_ref[...]) * di[:, None]


def _mid_layer(degp_ref, agg_ref, hp_ref, b_ref, w_ref, out_ref):
    di = _dinv(degp_ref[...])
    a = agg_ref[0] + agg_ref[1] + hp_ref[...]
    x = jnp.tanh(a * di[:, None] + b_ref[...])
    out_ref[...] = _mm(x, w_ref[...]) * di[:, None]


def _final_layer(degp_ref, agg_ref, hp_ref, b_ref, wc_ref, bc_ref, out_ref):
    di = _dinv(degp_ref[...])
    a = agg_ref[0] + agg_ref[1] + hp_ref[...]
    x = jnp.tanh(a * di[:, None] + b_ref[...])
    out_ref[...] = _mm(x, wc_ref[...]) + bc_ref[...]


def _row_spec(D):
    return pl.BlockSpec((_BR, D), lambda i: (i, 0))


def _agg_spec(D):
    return pl.BlockSpec((_NC, _BR, D), lambda i: (0, i, 0))


def _full_spec(shape):
    return pl.BlockSpec(shape, lambda i: tuple(0 for _ in shape))


def _tc_pre(degp, x, w):
    return pl.pallas_call(
        _first_layer_pre,
        grid=(_GRID,),
        in_specs=[_agg_spec(16), _row_spec(128), _full_spec((128, 128))],
        out_specs=_row_spec(128),
        out_shape=jax.ShapeDtypeStruct((_NP, 128), jnp.float32),
    )(degp, x, w)


def _tc_mid(degp, aggp, hp, b, w, dout):
    din = hp.shape[1]
    return pl.pallas_call(
        _mid_layer,
        grid=(_GRID,),
        in_specs=[_agg_spec(16), _agg_spec(din), _row_spec(din),
                  _full_spec((1, din)), _full_spec((din, dout))],
        out_specs=_row_spec(dout),
        out_shape=jax.ShapeDtypeStruct((_NP, dout), jnp.float32),
    )(degp, aggp, hp, b, w)


def _tc_final(degp, aggp, hp, b, wc, bc):
    return pl.pallas_call(
        _final_layer,
        grid=(_GRID,),
        in_specs=[_agg_spec(16), _agg_spec(128), _row_spec(128),
                  _full_spec((1, 128)), _full_spec((128, 128)),
                  _full_spec((1, 128))],
        out_specs=_row_spec(128),
        out_shape=jax.ShapeDtypeStruct((_NP, 128), jnp.float32),
    )(degp, aggp, hp, b, wc, bc)


_deg_kernel = _make_deg()
_agg128 = _make_agg(128)


def kernel(x, edge_index, W1, b1, W2, b2, W3, b3, Wc, bc):
    npad = _EPAD - _E
    src = jnp.concatenate([edge_index[0], jnp.zeros((npad,), jnp.int32)])
    dst = jnp.concatenate([edge_index[1],
                           jnp.full((npad,), _NP - 1, jnp.int32)])
    xp = jnp.pad(x, ((0, _NP - _N), (0, 0)))

    degp = _deg_kernel(dst)
    hp1 = _tc_pre(degp, xp, W1)
    a1 = _agg128(hp1, src, dst)
    hp2 = _tc_mid(degp, a1, hp1, b1.reshape(1, -1), W2, 128)
    a2 = _agg128(hp2, src, dst)
    w3p = jnp.pad(W3, ((0, 0), (0, 128 - W3.shape[1])))
    b3p = jnp.pad(b3, (0, 128 - b3.shape[0]))
    hp3 = _tc_mid(degp, a2, hp2, b2.reshape(1, -1), w3p, 128)
    a3 = _agg128(hp3, src, dst)

    wcp = jnp.pad(Wc, ((0, 128 - Wc.shape[0]), (0, 128 - Wc.shape[1])))
    bcp = jnp.pad(bc, (0, 128 - bc.shape[0])).reshape(1, -1)
    out = _tc_final(degp, a3, hp3, b3p.reshape(1, -1), wcp, bcp)
    return out[:_N, :Wc.shape[1]]


# speedup vs baseline: 2.4808x; 2.4808x over previous
import functools

import jax
import jax.numpy as jnp
from jax import lax
from jax.experimental import pallas as pl
from jax.experimental.pallas import tpu as pltpu
from jax.experimental.pallas import tpu_sc as plsc

_N = 10000
_E = 320000
_NP = 10240
_NC = 2
_NS = 16
_NW = _NC * _NS
_EW = _E // _NW
_CHUNK = 80
_NCHUNK = _EW // _CHUNK
_RSUB = _NP // _NS
_ZROWS = 32
_ZN = _RSUB // _ZROWS
_WROWS = 128
_WB = _RSUB // _WROWS


def _sc_mesh():
    return plsc.VectorSubcoreMesh(core_axis_name="c", subcore_axis_name="s")


def _make_agg(D):
    @functools.partial(
        pl.kernel,
        out_type=jax.ShapeDtypeStruct((_NC, _NP, D), jnp.float32),
        mesh=_sc_mesh(),
        scratch_types=[
            pltpu.VMEM((_CHUNK,), jnp.int32),
            pltpu.VMEM((_CHUNK,), jnp.int32),
            pltpu.VMEM((_CHUNK,), jnp.int32),
            pltpu.VMEM((_CHUNK,), jnp.int32),
            pltpu.VMEM((_CHUNK,), jnp.int32),
            pltpu.VMEM((_CHUNK,), jnp.int32),
            pltpu.VMEM((_CHUNK, D), jnp.float32),
            pltpu.VMEM((_CHUNK, D), jnp.float32),
            pltpu.VMEM((_CHUNK, D), jnp.float32),
            pltpu.VMEM((_ZROWS, D), jnp.float32),
            pltpu.VMEM_SHARED((_NP, D), jnp.float32),
            pltpu.SemaphoreType.DMA,
            pltpu.SemaphoreType.DMA,
            pltpu.SemaphoreType.DMA,
            pltpu.SemaphoreType.DMA,
            pltpu.SemaphoreType.DMA,
            pltpu.SemaphoreType.DMA,
        ],
    )
    def agg(hp_hbm, src_hbm, dst_hbm, out_hbm, s0, s1, s2, d0, d1, d2,
            m0, m1, m2, zv, acc, semi0, semi1, semi2, semg0, semg1, semg2):
        c = lax.axis_index("c")
        s = lax.axis_index("s")
        base = (c * _NS + s) * _EW
        sv = [s0, s1, s2]
        dv = [d0, d1, d2]
        mv = [m0, m1, m2]
        semi = [semi0, semi1, semi2]
        semg = [semg0, semg1, semg2]

        def ldx(i, j):
            off = base + i * _CHUNK
            ca = pltpu.make_async_copy(src_hbm.at[pl.ds(off, _CHUNK)], sv[j],
                                       semi[j])
            cb = pltpu.make_async_copy(dst_hbm.at[pl.ds(off, _CHUNK)], dv[j],
                                       semi[j])
            return ca, cb

        def ldx_start(i, j):
            ca, cb = ldx(i, j)
            ca.start()
            cb.start()

        def ldx_wait(i, j):
            ca, cb = ldx(i, j)
            ca.wait()
            cb.wait()

        def gather(j):
            return pltpu.make_async_copy(hp_hbm.at[sv[j]], mv[j], semg[j])

        def scat(j):
            pltpu.sync_copy(mv[j], acc.at[dv[j]], add=True)

        ldx_start(0, 0)
        ldx_start(1, 1)
        ldx_start(2, 2)

        @pl.loop(0, _ZROWS)
        def _zero(i):
            for j in range(D // 16):
                zv[i, pl.ds(j * 16, 16)] = jnp.zeros((16,), jnp.float32)

        @pl.loop(0, _ZN)
        def _clear(k):
            pltpu.sync_copy(zv, acc.at[pl.ds(s * _RSUB + k * _ZROWS, _ZROWS)])

        plsc.subcore_barrier()

        ldx_wait(0, 0)
        gather(0).start()
        ldx_wait(1, 1)
        gather(1).start()

        @pl.loop(0, (_NCHUNK - 5) // 3)
        def _edges(t):
            i = t * 3
            ldx_wait(i + 2, 2)
            gather(2).start()
            gather(0).wait()
            scat(0)
            ldx_start(i + 3, 0)
            ldx_wait(i + 3, 0)
            gather(0).start()
            gather(1).wait()
            scat(1)
            ldx_start(i + 4, 1)
            ldx_wait(i + 4, 1)
            gather(1).start()
            gather(2).wait()
            scat(2)
            ldx_start(i + 5, 2)

        i0 = _NCHUNK - 5
        ldx_wait(i0 + 2, 2)
        gather(2).start()
        gather(0).wait()
        scat(0)
        ldx_start(i0 + 3, 0)
        ldx_wait(i0 + 3, 0)
        gather(0).start()
        gather(1).wait()
        scat(1)
        ldx_start(i0 + 4, 1)
        ldx_wait(i0 + 4, 1)
        gather(1).start()
        gather(2).wait()
        scat(2)
        gather(0).wait()
        scat(0)
        gather(1).wait()
        scat(1)

        plsc.subcore_barrier()

        @pl.loop(0, _WB)
        def _writeback(k):
            r0 = s * _RSUB + k * _WROWS
            pltpu.sync_copy(acc.at[pl.ds(r0, _WROWS)],
                            out_hbm.at[c, pl.ds(r0, _WROWS)])

    return agg


def _make_deg():
    D = 16

    @functools.partial(
        pl.kernel,
        out_type=jax.ShapeDtypeStruct((_NC, _NP, D), jnp.float32),
        mesh=_sc_mesh(),
        scratch_types=[
            pltpu.VMEM((_CHUNK,), jnp.int32),
            pltpu.VMEM((_CHUNK,), jnp.int32),
            pltpu.VMEM((_CHUNK, D), jnp.float32),
            pltpu.VMEM((_ZROWS, D), jnp.float32),
            pltpu.VMEM_SHARED((_NP, D), jnp.float32),
            pltpu.SemaphoreType.DMA,
            pltpu.SemaphoreType.DMA,
            pltpu.SemaphoreType.DMA,
            pltpu.SemaphoreType.DMA,
        ],
    )
    def deg(dst_hbm, out_hbm, da, db, onesv, zv, acc, semia, semib, sema,
            semb):
        c = lax.axis_index("c")
        s = lax.axis_index("s")
        base = (c * _NS + s) * _EW

        def ldx(i, dv, sem):
            off = base + i * _CHUNK
            return pltpu.make_async_copy(dst_hbm.at[pl.ds(off, _CHUNK)], dv,
                                         sem)

        def scat(dv, sem):
            return pltpu.make_async_copy(onesv, acc.at[dv], sem)

        ldx(0, da, semia).start()

        @pl.loop(0, _ZROWS)
        def _zero(i):
            zv[i, pl.ds(0, 16)] = jnp.zeros((16,), jnp.float32)

        @pl.loop(0, _CHUNK)
        def _fill(i):
            onesv[i, pl.ds(0, 16)] = jnp.full((16,), 1.0, jnp.float32)

        @pl.loop(0, _ZN)
        def _clear(k):
            pltpu.sync_copy(zv, acc.at[pl.ds(s * _RSUB + k * _ZROWS, _ZROWS)])

        plsc.subcore_barrier()

        ldx(0, da, semia).wait()
        scat(da, sema).start(add=True)
        ldx(1, db, semib).start()

        @pl.loop(0, (_NCHUNK - 3) // 2)
        def _edges(k):
            i = k * 2
            ldx(i + 1, db, semib).wait()
            scat(db, semb).start(add=True)
            scat(da, sema).wait()
            ldx(i + 2, da, semia).start()
            ldx(i + 2, da, semia).wait()
            scat(da, sema).start(add=True)
            scat(db, semb).wait()
            ldx(i + 3, db, semib).start()

        i0 = _NCHUNK - 3
        ldx(i0 + 1, db, semib).wait()
        scat(db, semb).start(add=True)
        scat(da, sema).wait()
        ldx(i0 + 2, da, semia).start()
        ldx(i0 + 2, da, semia).wait()
        scat(da, sema).start(add=True)
        scat(db, semb).wait()
        scat(da, sema).wait()

        plsc.subcore_barrier()

        @pl.loop(0, _WB)
        def _writeback(k):
            r0 = s * _RSUB + k * _WROWS
            pltpu.sync_copy(acc.at[pl.ds(r0, _WROWS)],
                            out_hbm.at[c, pl.ds(r0, _WROWS)])

    return deg


_BR = 512
_GRID = _NP // _BR


def _dinv(degp_blk):
    deg = degp_blk[0, :, 0] + degp_blk[1, :, 0] + 1.0
    return lax.rsqrt(deg)


def _mm(a, b):
    return jnp.dot(a, b, preferred_element_type=jnp.float32,
                   precision=lax.Precision.HIGHEST)


def _first_layer_pre(degp_ref, x_ref, w_ref, hp_ref):
    di = _dinv(degp_ref[...])
    hp_ref[...] = _mm(x_ref[...], w_ref[...]) * di[:, None]


def _mid_layer(degp_ref, agg_ref, hp_ref, b_ref, w_ref, out_ref):
    di = _dinv(degp_ref[...])
    a = agg_ref[0] + agg_ref[1] + hp_ref[...]
    x = jnp.tanh(a * di[:, None] + b_ref[...])
    out_ref[...] = _mm(x, w_ref[...]) * di[:, None]


def _final_layer(degp_ref, agg_ref, hp_ref, b_ref, wc_ref, bc_ref, out_ref):
    di = _dinv(degp_ref[...])
    a = agg_ref[0] + agg_ref[1] + hp_ref[...]
    x = jnp.tanh(a * di[:, None] + b_ref[...])
    out_ref[...] = _mm(x, wc_ref[...]) + bc_ref[...]


def _row_spec(D):
    return pl.BlockSpec((_BR, D), lambda i: (i, 0))


def _agg_spec(D):
    return pl.BlockSpec((_NC, _BR, D), lambda i: (0, i, 0))


def _full_spec(shape):
    return pl.BlockSpec(shape, lambda i: tuple(0 for _ in shape))


def _tc_pre(degp, x, w):
    return pl.pallas_call(
        _first_layer_pre,
        grid=(_GRID,),
        in_specs=[_agg_spec(16), _row_spec(128), _full_spec((128, 128))],
        out_specs=_row_spec(128),
        out_shape=jax.ShapeDtypeStruct((_NP, 128), jnp.float32),
    )(degp, x, w)


def _tc_mid(degp, aggp, hp, b, w, dout):
    din = hp.shape[1]
    return pl.pallas_call(
        _mid_layer,
        grid=(_GRID,),
        in_specs=[_agg_spec(16), _agg_spec(din), _row_spec(din),
                  _full_spec((1, din)), _full_spec((din, dout))],
        out_specs=_row_spec(dout),
        out_shape=jax.ShapeDtypeStruct((_NP, dout), jnp.float32),
    )(degp, aggp, hp, b, w)


def _tc_final(degp, aggp, hp, b, wc, bc):
    return pl.pallas_call(
        _final_layer,
        grid=(_GRID,),
        in_specs=[_agg_spec(16), _agg_spec(128), _row_spec(128),
                  _full_spec((1, 128)), _full_spec((128, 128)),
                  _full_spec((1, 128))],
        out_specs=_row_spec(128),
        out_shape=jax.ShapeDtypeStruct((_NP, 128), jnp.float32),
    )(degp, aggp, hp, b, wc, bc)


_deg_kernel = _make_deg()
_agg128 = _make_agg(128)


def kernel(x, edge_index, W1, b1, W2, b2, W3, b3, Wc, bc):
    src = edge_index[0]
    dst = edge_index[1]
    xp = jnp.pad(x, ((0, _NP - _N), (0, 0)))

    degp = _deg_kernel(dst)
    hp1 = _tc_pre(degp, xp, W1)
    a1 = _agg128(hp1, src, dst)
    hp2 = _tc_mid(degp, a1, hp1, b1.reshape(1, -1), W2, 128)
    a2 = _agg128(hp2, src, dst)
    w3p = jnp.pad(W3, ((0, 0), (0, 128 - W3.shape[1])))
    b3p = jnp.pad(b3, (0, 128 - b3.shape[0]))
    hp3 = _tc_mid(degp, a2, hp2, b2.reshape(1, -1), w3p, 128)
    a3 = _agg128(hp3, src, dst)

    wcp = jnp.pad(Wc, ((0, 128 - Wc.shape[0]), (0, 128 - Wc.shape[1])))
    bcp = jnp.pad(bc, (0, 128 - bc.shape[0])).reshape(1, -1)
    out = _tc_final(degp, a3, hp3, b3p.reshape(1, -1), wcp, bcp)
    return out[:_N, :Wc.shape[1]]


# speedup vs baseline: 2.7185x; 1.0958x over previous
import functools

import jax
import jax.numpy as jnp
from jax import lax
from jax.experimental import pallas as pl
from jax.experimental.pallas import tpu as pltpu
from jax.experimental.pallas import tpu_sc as plsc

_N = 10000
_E = 320000
_NP = 10240
_NC = 2
_NS = 16
_NW = _NC * _NS
_EW = _E // _NW
_CHUNK = 80
_NCHUNK = _EW // _CHUNK
_G = 5
_NG = _NCHUNK // _G
_RSUB = _NP // _NS
_ZROWS = 32
_ZN = _RSUB // _ZROWS
_WROWS = 128
_WB = _RSUB // _WROWS


def _sc_mesh():
    return plsc.VectorSubcoreMesh(core_axis_name="c", subcore_axis_name="s")


def _make_agg(D):
    @functools.partial(
        pl.kernel,
        out_type=jax.ShapeDtypeStruct((_NC, _NP, D), jnp.float32),
        mesh=_sc_mesh(),
        scratch_types=[
            pltpu.VMEM((_G * _CHUNK,), jnp.int32),
            pltpu.VMEM((_G, _CHUNK), jnp.int32),
            pltpu.VMEM((_G * _CHUNK,), jnp.int32),
            pltpu.VMEM((_G, _CHUNK), jnp.int32),
            pltpu.VMEM((_CHUNK, D), jnp.float32),
            pltpu.VMEM((_CHUNK, D), jnp.float32),
            pltpu.VMEM((_ZROWS, D), jnp.float32),
            pltpu.VMEM_SHARED((_NP, D), jnp.float32),
            pltpu.SemaphoreType.DMA,
            pltpu.SemaphoreType.DMA,
            pltpu.SemaphoreType.DMA,
            pltpu.SemaphoreType.DMA,
        ],
    )
    def agg(hp_hbm, src_hbm, dst3_hbm, out_hbm, sa, da, sb, db, ma, mb, zv,
            acc, semia, semib, sema, semb):
        c = lax.axis_index("c")
        s = lax.axis_index("s")
        w = c * _NS + s
        base = w * _EW
        seta = (sa, da, semia)
        setb = (sb, db, semib)

        def ldg(g, st):
            sv, dv, sem = st
            ca = pltpu.make_async_copy(
                src_hbm.at[pl.ds(base + g * _G * _CHUNK, _G * _CHUNK)], sv,
                sem)
            cb = pltpu.make_async_copy(dst3_hbm.at[w * _NG + g], dv, sem)
            return ca, cb

        def ldg_start(g, st):
            ca, cb = ldg(g, st)
            ca.start()
            cb.start()

        def ldg_wait(g, st):
            ca, cb = ldg(g, st)
            ca.wait()
            cb.wait()

        def gath(st, j, buf, sem):
            sv = st[0]
            return pltpu.make_async_copy(
                hp_hbm.at[sv.at[pl.ds(j * _CHUNK, _CHUNK)]], buf, sem)

        ldg_start(0, seta)

        @pl.loop(0, _ZROWS)
        def _zero(i):
            for j in range(D // 16):
                zv[i, pl.ds(j * 16, 16)] = jnp.zeros((16,), jnp.float32)

        @pl.loop(0, _ZN)
        def _clear(k):
            pltpu.sync_copy(zv, acc.at[pl.ds(s * _RSUB + k * _ZROWS, _ZROWS)])

        plsc.subcore_barrier()

        def group(g, st, first_ma, nxt, next_gather, next_ld):
            bufs = (ma, mb) if first_ma else (mb, ma)
            sems = (sema, semb) if first_ma else (semb, sema)
            for j in range(_G - 1):
                gath(st, j + 1, bufs[(j + 1) % 2], sems[(j + 1) % 2]).start()
                gath(st, j, bufs[j % 2], sems[j % 2]).wait()
                pltpu.sync_copy(bufs[j % 2], acc.at[st[1].at[j]], add=True)
            if next_gather:
                ldg_wait(g + 1, nxt)
                gath(nxt, 0, bufs[1], sems[1]).start()
            gath(st, _G - 1, bufs[0], sems[0]).wait()
            pltpu.sync_copy(bufs[0], acc.at[st[1].at[_G - 1]], add=True)
            if next_ld:
                ldg_start(g + 2, st)

        ldg_wait(0, seta)
        gath(seta, 0, ma, sema).start()
        ldg_start(1, setb)

        @pl.loop(0, (_NG - 3) // 2)
        def _edges(kp):
            g = kp * 2
            group(g, seta, True, setb, True, True)
            group(g + 1, setb, False, seta, True, True)

        g0 = _NG - 3
        group(g0, seta, True, setb, True, True)
        group(g0 + 1, setb, False, seta, True, False)
        group(g0 + 2, seta, True, setb, False, False)

        plsc.subcore_barrier()

        @pl.loop(0, _WB)
        def _writeback(k):
            r0 = s * _RSUB + k * _WROWS
            pltpu.sync_copy(acc.at[pl.ds(r0, _WROWS)],
                            out_hbm.at[c, pl.ds(r0, _WROWS)])

    return agg


def _make_deg():
    D = 16

    @functools.partial(
        pl.kernel,
        out_type=jax.ShapeDtypeStruct((_NC, _NP, D), jnp.float32),
        mesh=_sc_mesh(),
        scratch_types=[
            pltpu.VMEM((_CHUNK,), jnp.int32),
            pltpu.VMEM((_CHUNK,), jnp.int32),
            pltpu.VMEM((_CHUNK, D), jnp.float32),
            pltpu.VMEM((_ZROWS, D), jnp.float32),
            pltpu.VMEM_SHARED((_NP, D), jnp.float32),
            pltpu.SemaphoreType.DMA,
            pltpu.SemaphoreType.DMA,
            pltpu.SemaphoreType.DMA,
            pltpu.SemaphoreType.DMA,
        ],
    )
    def deg(dst_hbm, out_hbm, da, db, onesv, zv, acc, semia, semib, sema,
            semb):
        c = lax.axis_index("c")
        s = lax.axis_index("s")
        base = (c * _NS + s) * _EW

        def ldx(i, dv, sem):
            off = base + i * _CHUNK
            return pltpu.make_async_copy(dst_hbm.at[pl.ds(off, _CHUNK)], dv,
                                         sem)

        def scat(dv, sem):
            return pltpu.make_async_copy(onesv, acc.at[dv], sem)

        ldx(0, da, semia).start()

        @pl.loop(0, _ZROWS)
        def _zero(i):
            zv[i, pl.ds(0, 16)] = jnp.zeros((16,), jnp.float32)

        @pl.loop(0, _CHUNK)
        def _fill(i):
            onesv[i, pl.ds(0, 16)] = jnp.full((16,), 1.0, jnp.float32)

        @pl.loop(0, _ZN)
        def _clear(k):
            pltpu.sync_copy(zv, acc.at[pl.ds(s * _RSUB + k * _ZROWS, _ZROWS)])

        plsc.subcore_barrier()

        ldx(0, da, semia).wait()
        scat(da, sema).start(add=True)
        ldx(1, db, semib).start()

        @pl.loop(0, (_NCHUNK - 3) // 2)
        def _edges(k):
            i = k * 2
            ldx(i + 1, db, semib).wait()
            scat(db, semb).start(add=True)
            scat(da, sema).wait()
            ldx(i + 2, da, semia).start()
            ldx(i + 2, da, semia).wait()
            scat(da, sema).start(add=True)
            scat(db, semb).wait()
            ldx(i + 3, db, semib).start()

        i0 = _NCHUNK - 3
        ldx(i0 + 1, db, semib).wait()
        scat(db, semb).start(add=True)
        scat(da, sema).wait()
        ldx(i0 + 2, da, semia).start()
        ldx(i0 + 2, da, semia).wait()
        scat(da, sema).start(add=True)
        scat(db, semb).wait()
        scat(da, sema).wait()

        plsc.subcore_barrier()

        @pl.loop(0, _WB)
        def _writeback(k):
            r0 = s * _RSUB + k * _WROWS
            pltpu.sync_copy(acc.at[pl.ds(r0, _WROWS)],
                            out_hbm.at[c, pl.ds(r0, _WROWS)])

    return deg


_BR = 512
_GRID = _NP // _BR


def _dinv(degp_blk):
    deg = degp_blk[0, :, 0] + degp_blk[1, :, 0] + 1.0
    return lax.rsqrt(deg)


def _mm(a, b):
    return jnp.dot(a, b, preferred_element_type=jnp.float32,
                   precision=lax.Precision.HIGHEST)


def _first_layer_pre(degp_ref, x_ref, w_ref, hp_ref):
    di = _dinv(degp_ref[...])
    hp_ref[...] = _mm(x_ref[...], w_ref[...]) * di[:, None]


def _mid_layer(degp_ref, agg_ref, hp_ref, b_ref, w_ref, out_ref):
    di = _dinv(degp_ref[...])
    a = agg_ref[0] + agg_ref[1] + hp_ref[...]
    x = jnp.tanh(a * di[:, None] + b_ref[...])
    out_ref[...] = _mm(x, w_ref[...]) * di[:, None]


def _final_layer(degp_ref, agg_ref, hp_ref, b_ref, wc_ref, bc_ref, out_ref):
    di = _dinv(degp_ref[...])
    a = agg_ref[0] + agg_ref[1] + hp_ref[...]
    x = jnp.tanh(a * di[:, None] + b_ref[...])
    out_ref[...] = _mm(x, wc_ref[...]) + bc_ref[...]


def _row_spec(D):
    return pl.BlockSpec((_BR, D), lambda i: (i, 0))


def _agg_spec(D):
    return pl.BlockSpec((_NC, _BR, D), lambda i: (0, i, 0))


def _full_spec(shape):
    return pl.BlockSpec(shape, lambda i: tuple(0 for _ in shape))


def _tc_pre(degp, x, w):
    return pl.pallas_call(
        _first_layer_pre,
        grid=(_GRID,),
        in_specs=[_agg_spec(16), _row_spec(128), _full_spec((128, 128))],
        out_specs=_row_spec(128),
        out_shape=jax.ShapeDtypeStruct((_NP, 128), jnp.float32),
    )(degp, x, w)


def _tc_mid(degp, aggp, hp, b, w, dout):
    din = hp.shape[1]
    return pl.pallas_call(
        _mid_layer,
        grid=(_GRID,),
        in_specs=[_agg_spec(16), _agg_spec(din), _row_spec(din),
                  _full_spec((1, din)), _full_spec((din, dout))],
        out_specs=_row_spec(dout),
        out_shape=jax.ShapeDtypeStruct((_NP, dout), jnp.float32),
    )(degp, aggp, hp, b, w)


def _tc_final(degp, aggp, hp, b, wc, bc):
    return pl.pallas_call(
        _final_layer,
        grid=(_GRID,),
        in_specs=[_agg_spec(16), _agg_spec(128), _row_spec(128),
                  _full_spec((1, 128)), _full_spec((128, 128)),
                  _full_spec((1, 128))],
        out_specs=_row_spec(128),
        out_shape=jax.ShapeDtypeStruct((_NP, 128), jnp.float32),
    )(degp, aggp, hp, b, wc, bc)


_deg_kernel = _make_deg()
_agg128 = _make_agg(128)


def kernel(x, edge_index, W1, b1, W2, b2, W3, b3, Wc, bc):
    src = edge_index[0]
    dst = edge_index[1]
    dst3 = dst.reshape(_NW * _NG, _G, _CHUNK)
    xp = jnp.pad(x, ((0, _NP - _N), (0, 0)))

    degp = _deg_kernel(dst)
    hp1 = _tc_pre(degp, xp, W1)
    a1 = _agg128(hp1, src, dst3)
    hp2 = _tc_mid(degp, a1, hp1, b1.reshape(1, -1), W2, 128)
    a2 = _agg128(hp2, src, dst3)
    w3p = jnp.pad(W3, ((0, 0), (0, 128 - W3.shape[1])))
    b3p = jnp.pad(b3, (0, 128 - b3.shape[0]))
    hp3 = _tc_mid(degp, a2, hp2, b2.reshape(1, -1), w3p, 128)
    a3 = _agg128(hp3, src, dst3)

    wcp = jnp.pad(Wc, ((0, 128 - Wc.shape[0]), (0, 128 - Wc.shape[1])))
    bcp = jnp.pad(bc, (0, 128 - bc.shape[0])).reshape(1, -1)
    out = _tc_final(degp, a3, hp3, b3p.reshape(1, -1), wcp, bcp)
    return out[:_N, :Wc.shape[1]]
